# baseline (speedup 1.0000x reference)
import jax
import jax.numpy as jnp
from jax import lax
from jax.experimental import pallas as pl
from jax.experimental.pallas import tpu as pltpu

F = 2
GROUPS = 8


def _top2(f):
    rows, cols = f.shape
    iota = lax.broadcasted_iota(jnp.int32, (rows, cols), 1)
    m1 = jnp.max(f, axis=1, keepdims=True)
    i1 = jnp.min(jnp.where(f == m1, iota, cols), axis=1, keepdims=True)
    f2 = jnp.where(iota == i1, -jnp.inf, f)
    m2 = jnp.max(f2, axis=1, keepdims=True)
    i2 = jnp.min(jnp.where(f2 == m2, iota, cols), axis=1, keepdims=True)
    return i1, i2, m1, m2, iota


def _attn_body(focus_ref, dec_h_ref, bdec_ref, wv_ref, wcov_ref,
               wd_hbm, ef_hbm, eo_hbm, cov_hbm, mask_hbm,
               ctx_ref, attn_ref, covout_ref,
               efg_ref, eog_ref, wd_ref, cov_ref, mask_ref,
               iv_ref, is_ref, sems_ef, sems_eo, sem_w, isem):
    bsz, secl, wordl = cov_ref.shape
    gb = bsz // GROUPS

    i1, i2, m1, m2, iota_s = _top2(focus_ref[...])
    iv_ref[...] = jnp.concatenate([i1, i2], axis=1)
    idx_copy = pltpu.make_async_copy(iv_ref, is_ref, isem)
    idx_copy.start()

    wd_copy = pltpu.make_async_copy(wd_hbm, wd_ref, sem_w)
    cov_copy = pltpu.make_async_copy(cov_hbm, cov_ref, sem_w)
    mask_copy = pltpu.make_async_copy(mask_hbm, mask_ref, sem_w)
    wd_copy.start()
    cov_copy.start()
    mask_copy.start()

    oh0 = (iota_s == i1).astype(jnp.float32)
    oh1 = (iota_s == i2).astype(jnp.float32)

    idx_copy.wait()

    ef_copies = [[] for _ in range(GROUPS)]
    eo_copies = [[] for _ in range(GROUPS)]
    for b in range(bsz):
        g = b // gb
        for f in range(F):
            sec = is_ref[b, f]
            ef_copies[g].append(pltpu.make_async_copy(
                ef_hbm.at[b, sec], efg_ref.at[b, pl.ds(f * wordl, wordl), :],
                sems_ef.at[g]))
            eo_copies[g].append(pltpu.make_async_copy(
                eo_hbm.at[b, sec], eog_ref.at[b, pl.ds(f * wordl, wordl), :],
                sems_eo.at[g]))
    for grp in ef_copies:
        for c in grp:
            c.start()
    for grp in eo_copies:
        for c in grp:
            c.start()

    wd_copy.wait()
    dec = lax.dot_general(
        dec_h_ref[...], wd_ref[...], (((1,), (1,)), ((), ())),
        preferred_element_type=jnp.float32)
    dec = dec + bdec_ref[...]

    cov_copy.wait()
    mask_copy.wait()
    mask3 = mask_ref[...]
    cov3 = cov_ref[...]
    mask_row = jnp.concatenate(
        [jnp.sum(oh0[:, :, None] * mask3, axis=1),
         jnp.sum(oh1[:, :, None] * mask3, axis=1)], axis=1)
    cov_row = jnp.concatenate(
        [jnp.sum(oh0[:, :, None] * cov3, axis=1),
         jnp.sum(oh1[:, :, None] * cov3, axis=1)], axis=1)
    foc_row = jnp.concatenate(
        [jnp.broadcast_to(m1, (bsz, wordl)),
         jnp.broadcast_to(m2, (bsz, wordl))], axis=1)

    wv = wv_ref[...]
    wcov = wcov_ref[...]

    for g in range(GROUPS):
        for c in ef_copies[g]:
            c.wait()
        sl = pl.ds(g * gb, gb)
        x = (efg_ref[sl] + dec[g * gb:(g + 1) * gb, None, :]
             + cov_row[g * gb:(g + 1) * gb, :, None] * wcov[None, :, :])
        t = jnp.tanh(x)
        s = jnp.sum(t * wv[None, :, :], axis=2)

        e = jnp.exp(s - jnp.max(s, axis=1, keepdims=True))
        af = e * mask_row[g * gb:(g + 1) * gb] * foc_row[g * gb:(g + 1) * gb]
        w = af / jnp.sum(af, axis=1, keepdims=True)

        for c in eo_copies[g]:
            c.wait()
        ctx_ref[sl] = jnp.sum(w[:, :, None] * eog_ref[sl], axis=1)

        attn = (oh0[g * gb:(g + 1) * gb, :, None] * w[:, None, :wordl]
                + oh1[g * gb:(g + 1) * gb, :, None] * w[:, None, wordl:])
        attn_ref[sl] = attn
        covout_ref[sl] = cov3[g * gb:(g + 1) * gb] + attn


def kernel(dec_hidden, enc_output, enc_feature, enc_mask, sec_attn, coverage,
           focus, W_dec, b_dec, w_v, w_cov):
    batch, src_len, dim = enc_output.shape
    secl = focus.shape[1]
    wordl = src_len // secl

    ef = enc_feature.reshape(batch, secl, wordl, dim)
    eo = enc_output.reshape(batch, secl, wordl, dim)
    cov3 = coverage.reshape(batch, secl, wordl)
    mask3 = enc_mask.reshape(batch, secl, wordl)

    context, attn3, covout3 = pl.pallas_call(
        _attn_body,
        in_specs=[
            pl.BlockSpec(memory_space=pltpu.VMEM),
            pl.BlockSpec(memory_space=pltpu.VMEM),
            pl.BlockSpec(memory_space=pltpu.VMEM),
            pl.BlockSpec(memory_space=pltpu.VMEM),
            pl.BlockSpec(memory_space=pltpu.VMEM),
            pl.BlockSpec(memory_space=pltpu.HBM),
            pl.BlockSpec(memory_space=pltpu.HBM),
            pl.BlockSpec(memory_space=pltpu.HBM),
            pl.BlockSpec(memory_space=pltpu.HBM),
            pl.BlockSpec(memory_space=pltpu.HBM),
        ],
        out_specs=[
            pl.BlockSpec(memory_space=pltpu.VMEM),
            pl.BlockSpec(memory_space=pltpu.VMEM),
            pl.BlockSpec(memory_space=pltpu.VMEM),
        ],
        scratch_shapes=[
            pltpu.VMEM((batch, F * wordl, dim), jnp.float32),
            pltpu.VMEM((batch, F * wordl, dim), jnp.float32),
            pltpu.VMEM((dim, dim), jnp.float32),
            pltpu.VMEM((batch, secl, wordl), jnp.float32),
            pltpu.VMEM((batch, secl, wordl), jnp.float32),
            pltpu.VMEM((batch, F), jnp.int32),
            pltpu.SMEM((batch, F), jnp.int32),
            pltpu.SemaphoreType.DMA((GROUPS,)),
            pltpu.SemaphoreType.DMA((GROUPS,)),
            pltpu.SemaphoreType.DMA,
            pltpu.SemaphoreType.DMA,
        ],
        out_shape=(jax.ShapeDtypeStruct((batch, dim), jnp.float32),
                   jax.ShapeDtypeStruct((batch, secl, wordl), jnp.float32),
                   jax.ShapeDtypeStruct((batch, secl, wordl), jnp.float32)),
    )(focus, dec_hidden, b_dec.reshape(1, dim),
      w_v.reshape(1, dim), w_cov.reshape(1, dim), W_dec, ef, eo, cov3, mask3)

    return (context, attn3.reshape(batch, src_len),
            covout3.reshape(batch, src_len))


# speedup vs baseline: 1.0329x; 1.0329x over previous
import jax
import jax.numpy as jnp
from jax import lax
from jax.experimental import pallas as pl
from jax.experimental.pallas import tpu as pltpu

F = 2
GROUPS = 4


def _top2(f):
    rows, cols = f.shape
    iota = lax.broadcasted_iota(jnp.int32, (rows, cols), 1)
    m1 = jnp.max(f, axis=1, keepdims=True)
    i1 = jnp.min(jnp.where(f == m1, iota, cols), axis=1, keepdims=True)
    f2 = jnp.where(iota == i1, -jnp.inf, f)
    m2 = jnp.max(f2, axis=1, keepdims=True)
    i2 = jnp.min(jnp.where(f2 == m2, iota, cols), axis=1, keepdims=True)
    return i1, i2, m1, m2, iota


def _attn_body(focus_ref, dec_h_ref, wd_ref, bdec_ref, wv_ref, wcov_ref,
               ef_hbm, eo_hbm, cov_ref, mask_ref,
               ctx_ref, attn_ref, covout_ref,
               efg0, efg1, efg2, efg3, eog0, eog1, eog2, eog3,
               iv_ref, is_ref, sems_ef, sems_eo, isem):
    bsz, secl, wordl = cov_ref.shape
    gb = bsz // GROUPS
    efgs = [efg0, efg1, efg2, efg3]
    eogs = [eog0, eog1, eog2, eog3]

    i1, i2, m1, m2, iota_s = _top2(focus_ref[...])
    iv_ref[...] = jnp.concatenate([i1, i2], axis=1)
    idx_copy = pltpu.make_async_copy(iv_ref, is_ref, isem)
    idx_copy.start()

    oh0 = (iota_s == i1).astype(jnp.float32)
    oh1 = (iota_s == i2).astype(jnp.float32)

    idx_copy.wait()

    ef_copies = [[] for _ in range(GROUPS)]
    eo_copies = [[] for _ in range(GROUPS)]
    for b in range(bsz):
        g = b // gb
        lb = b % gb
        for f in range(F):
            sec = is_ref[b, f]
            ef_copies[g].append(pltpu.make_async_copy(
                ef_hbm.at[b, sec],
                efgs[g].at[lb, pl.ds(f * wordl, wordl), :],
                sems_ef.at[g]))
            eo_copies[g].append(pltpu.make_async_copy(
                eo_hbm.at[b, sec],
                eogs[g].at[lb, pl.ds(f * wordl, wordl), :],
                sems_eo.at[g]))
    for g in range(GROUPS):
        for c in ef_copies[g]:
            c.start()
        for c in eo_copies[g]:
            c.start()

    dec = lax.dot_general(
        dec_h_ref[...], wd_ref[...], (((1,), (1,)), ((), ())),
        preferred_element_type=jnp.float32)
    dec = dec + bdec_ref[...]

    mask3 = mask_ref[...]
    cov3 = cov_ref[...]
    mask_row = jnp.concatenate(
        [jnp.sum(oh0[:, :, None] * mask3, axis=1),
         jnp.sum(oh1[:, :, None] * mask3, axis=1)], axis=1)
    cov_row = jnp.concatenate(
        [jnp.sum(oh0[:, :, None] * cov3, axis=1),
         jnp.sum(oh1[:, :, None] * cov3, axis=1)], axis=1)
    foc_row = jnp.concatenate(
        [jnp.broadcast_to(m1, (bsz, wordl)),
         jnp.broadcast_to(m2, (bsz, wordl))], axis=1)

    wv = wv_ref[...]
    wcov = wcov_ref[...]

    for g in range(GROUPS):
        for c in ef_copies[g]:
            c.wait()
        sl = pl.ds(g * gb, gb)
        x = (efgs[g][...] + dec[g * gb:(g + 1) * gb, None, :]
             + cov_row[g * gb:(g + 1) * gb, :, None] * wcov[None, :, :])
        t = jnp.tanh(x)
        s = jnp.sum(t * wv[None, :, :], axis=2)

        e = jnp.exp(s - jnp.max(s, axis=1, keepdims=True))
        af = e * mask_row[g * gb:(g + 1) * gb] * foc_row[g * gb:(g + 1) * gb]
        w = af / jnp.sum(af, axis=1, keepdims=True)

        for c in eo_copies[g]:
            c.wait()
        ctx_ref[sl] = jnp.sum(w[:, :, None] * eogs[g][...], axis=1)

        attn = (oh0[g * gb:(g + 1) * gb, :, None] * w[:, None, :wordl]
                + oh1[g * gb:(g + 1) * gb, :, None] * w[:, None, wordl:])
        attn_ref[sl] = attn
        covout_ref[sl] = cov3[g * gb:(g + 1) * gb] + attn


def kernel(dec_hidden, enc_output, enc_feature, enc_mask, sec_attn, coverage,
           focus, W_dec, b_dec, w_v, w_cov):
    batch, src_len, dim = enc_output.shape
    secl = focus.shape[1]
    wordl = src_len // secl
    gb = batch // GROUPS

    ef = enc_feature.reshape(batch, secl, wordl, dim)
    eo = enc_output.reshape(batch, secl, wordl, dim)
    cov3 = coverage.reshape(batch, secl, wordl)
    mask3 = enc_mask.reshape(batch, secl, wordl)

    context, attn3, covout3 = pl.pallas_call(
        _attn_body,
        in_specs=[
            pl.BlockSpec(memory_space=pltpu.VMEM),
            pl.BlockSpec(memory_space=pltpu.VMEM),
            pl.BlockSpec(memory_space=pltpu.VMEM),
            pl.BlockSpec(memory_space=pltpu.VMEM),
            pl.BlockSpec(memory_space=pltpu.VMEM),
            pl.BlockSpec(memory_space=pltpu.VMEM),
            pl.BlockSpec(memory_space=pltpu.HBM),
            pl.BlockSpec(memory_space=pltpu.HBM),
            pl.BlockSpec(memory_space=pltpu.VMEM),
            pl.BlockSpec(memory_space=pltpu.VMEM),
        ],
        out_specs=[
            pl.BlockSpec(memory_space=pltpu.VMEM),
            pl.BlockSpec(memory_space=pltpu.VMEM),
            pl.BlockSpec(memory_space=pltpu.VMEM),
        ],
        scratch_shapes=(
            [pltpu.VMEM((gb, F * wordl, dim), jnp.float32)
             for _ in range(2 * GROUPS)]
            + [pltpu.VMEM((batch, F), jnp.int32),
               pltpu.SMEM((batch, F), jnp.int32),
               pltpu.SemaphoreType.DMA((GROUPS,)),
               pltpu.SemaphoreType.DMA((GROUPS,)),
               pltpu.SemaphoreType.DMA]),
        out_shape=(jax.ShapeDtypeStruct((batch, dim), jnp.float32),
                   jax.ShapeDtypeStruct((batch, secl, wordl), jnp.float32),
                   jax.ShapeDtypeStruct((batch, secl, wordl), jnp.float32)),
    )(focus, dec_hidden, W_dec, b_dec.reshape(1, dim),
      w_v.reshape(1, dim), w_cov.reshape(1, dim), ef, eo, cov3, mask3)

    return (context, attn3.reshape(batch, src_len),
            covout3.reshape(batch, src_len))


# speedup vs baseline: 2.0498x; 1.9845x over previous
import jax
import jax.numpy as jnp
from jax import lax
from jax.experimental import pallas as pl
from jax.experimental.pallas import tpu as pltpu

F = 2
GROUPS = 4


def _top2(f):
    rows, cols = f.shape
    iota = lax.broadcasted_iota(jnp.int32, (rows, cols), 1)
    m1 = jnp.max(f, axis=1, keepdims=True)
    i1 = jnp.min(jnp.where(f == m1, iota, cols), axis=1, keepdims=True)
    f2 = jnp.where(iota == i1, -jnp.inf, f)
    m2 = jnp.max(f2, axis=1, keepdims=True)
    i2 = jnp.min(jnp.where(f2 == m2, iota, cols), axis=1, keepdims=True)
    return i1, i2, m1, m2, iota


def _attn_body(focus_ref, dec_h_ref, wd_ref, bdec_ref, wv_ref, wcov_ref,
               ef_hbm, eo_hbm, cov_ref, mask_ref,
               ctx_ref, attn_ref, covout_ref,
               efg0, efg1, efg2, efg3, eog0, eog1, eog2, eog3,
               iv_ref, is_ref, sems_ef, sems_eo, isem):
    bsz, src_len = cov_ref.shape
    secl = focus_ref.shape[1]
    wordl = src_len // secl
    gb = bsz // GROUPS
    efgs = [efg0, efg1, efg2, efg3]
    eogs = [eog0, eog1, eog2, eog3]

    i1, i2, m1, m2, _ = _top2(focus_ref[...])
    iv_ref[...] = jnp.concatenate([i1, i2], axis=1)
    idx_copy = pltpu.make_async_copy(iv_ref, is_ref, isem)
    idx_copy.start()

    pos = lax.broadcasted_iota(jnp.int32, (bsz, src_len), 1)
    sec_of_pos = pos // wordl
    oh0_full = (sec_of_pos == i1).astype(jnp.float32)
    oh1_full = (sec_of_pos == i2).astype(jnp.float32)

    tj = lax.broadcasted_iota(jnp.int32, (wordl, src_len), 0)
    tp = lax.broadcasted_iota(jnp.int32, (wordl, src_len), 1)
    t_sel = (tp % wordl == tj).astype(jnp.float32)

    idx_copy.wait()

    ef_copies = [[] for _ in range(GROUPS)]
    eo_copies = [[] for _ in range(GROUPS)]
    for b in range(bsz):
        g = b // gb
        lb = b % gb
        for f in range(F):
            sec = is_ref[b, f]
            ef_copies[g].append(pltpu.make_async_copy(
                ef_hbm.at[b, sec],
                efgs[g].at[lb, pl.ds(f * wordl, wordl), :],
                sems_ef.at[g]))
            eo_copies[g].append(pltpu.make_async_copy(
                eo_hbm.at[b, sec],
                eogs[g].at[lb, pl.ds(f * wordl, wordl), :],
                sems_eo.at[g]))
    for g in range(GROUPS):
        for c in ef_copies[g]:
            c.start()
        for c in eo_copies[g]:
            c.start()

    dec = lax.dot_general(
        dec_h_ref[...], wd_ref[...], (((1,), (1,)), ((), ())),
        preferred_element_type=jnp.float32)
    dec = dec + bdec_ref[...]

    cov2 = cov_ref[...]
    mask2 = mask_ref[...]

    def _rows(full2, oh_full):
        return lax.dot_general(
            full2 * oh_full, t_sel, (((1,), (1,)), ((), ())),
            preferred_element_type=jnp.float32)

    mask_row = jnp.concatenate(
        [_rows(mask2, oh0_full), _rows(mask2, oh1_full)], axis=1)
    cov_row = jnp.concatenate(
        [_rows(cov2, oh0_full), _rows(cov2, oh1_full)], axis=1)
    foc_row = jnp.concatenate(
        [jnp.broadcast_to(m1, (bsz, wordl)),
         jnp.broadcast_to(m2, (bsz, wordl))], axis=1)

    wv = wv_ref[...]
    wcov = wcov_ref[...]

    for g in range(GROUPS):
        for c in ef_copies[g]:
            c.wait()
        sl = pl.ds(g * gb, gb)
        x = (efgs[g][...] + dec[g * gb:(g + 1) * gb, None, :]
             + cov_row[g * gb:(g + 1) * gb, :, None] * wcov[None, :, :])
        t = jnp.tanh(x)
        s = jnp.sum(t * wv[None, :, :], axis=2)

        e = jnp.exp(s - jnp.max(s, axis=1, keepdims=True))
        af = e * mask_row[g * gb:(g + 1) * gb] * foc_row[g * gb:(g + 1) * gb]
        w = af / jnp.sum(af, axis=1, keepdims=True)

        for c in eo_copies[g]:
            c.wait()
        ctx_ref[sl] = jnp.sum(w[:, :, None] * eogs[g][...], axis=1)

        w_full0 = lax.dot_general(
            w[:, :wordl], t_sel, (((1,), (0,)), ((), ())),
            preferred_element_type=jnp.float32)
        w_full1 = lax.dot_general(
            w[:, wordl:], t_sel, (((1,), (0,)), ((), ())),
            preferred_element_type=jnp.float32)
        attn = (oh0_full[g * gb:(g + 1) * gb] * w_full0
                + oh1_full[g * gb:(g + 1) * gb] * w_full1)
        attn_ref[sl] = attn
        covout_ref[sl] = cov2[g * gb:(g + 1) * gb] + attn


def kernel(dec_hidden, enc_output, enc_feature, enc_mask, sec_attn, coverage,
           focus, W_dec, b_dec, w_v, w_cov):
    batch, src_len, dim = enc_output.shape
    secl = focus.shape[1]
    wordl = src_len // secl
    gb = batch // GROUPS

    ef = enc_feature.reshape(batch, secl, wordl, dim)
    eo = enc_output.reshape(batch, secl, wordl, dim)

    context, attn_dist, covout = pl.pallas_call(
        _attn_body,
        in_specs=[
            pl.BlockSpec(memory_space=pltpu.VMEM),
            pl.BlockSpec(memory_space=pltpu.VMEM),
            pl.BlockSpec(memory_space=pltpu.VMEM),
            pl.BlockSpec(memory_space=pltpu.VMEM),
            pl.BlockSpec(memory_space=pltpu.VMEM),
            pl.BlockSpec(memory_space=pltpu.VMEM),
            pl.BlockSpec(memory_space=pltpu.HBM),
            pl.BlockSpec(memory_space=pltpu.HBM),
            pl.BlockSpec(memory_space=pltpu.VMEM),
            pl.BlockSpec(memory_space=pltpu.VMEM),
        ],
        out_specs=[
            pl.BlockSpec(memory_space=pltpu.VMEM),
            pl.BlockSpec(memory_space=pltpu.VMEM),
            pl.BlockSpec(memory_space=pltpu.VMEM),
        ],
        scratch_shapes=(
            [pltpu.VMEM((gb, F * wordl, dim), jnp.float32)
             for _ in range(2 * GROUPS)]
            + [pltpu.VMEM((batch, F), jnp.int32),
               pltpu.SMEM((batch, F), jnp.int32),
               pltpu.SemaphoreType.DMA((GROUPS,)),
               pltpu.SemaphoreType.DMA((GROUPS,)),
               pltpu.SemaphoreType.DMA]),
        out_shape=(jax.ShapeDtypeStruct((batch, dim), jnp.float32),
                   jax.ShapeDtypeStruct((batch, src_len), jnp.float32),
                   jax.ShapeDtypeStruct((batch, src_len), jnp.float32)),
    )(focus, dec_hidden, W_dec, b_dec.reshape(1, dim),
      w_v.reshape(1, dim), w_cov.reshape(1, dim), ef, eo, coverage, enc_mask)

    return (context, attn_dist, covout)
